# baseline (speedup 1.0000x reference)
import functools

import numpy as np
import jax
import jax.numpy as jnp
from jax import lax
from jax.experimental import pallas as pl
from jax.experimental.pallas import tpu as pltpu
from jax.experimental.pallas import tpu_sc as plsc

_Z_MIN = -2.0
_Z_STEP = 0.2
_Z_DEPTH = 100
_FOV = 2.268
_NA = 192
_R_MIN = 2.7
_R_MAX = 165.0
_NR = 320

_N_PTS = 5 * 131072
_GRID = 500 * _NA * _NR
_HALF = _GRID // 2
_ROW = _NA * _NR

_INVALID = np.int32(1 << 30)

_PTS_PER_TILE = _N_PTS // 16
_CHUNK = 8192
_N_CHUNKS = _PTS_PER_TILE // _CHUNK
_SURV_CAP = 2048
_ZSLAB = _HALF // 16
_ZBUF = 48000
_NZDMA = _ZSLAB // _ZBUF


def _build_tables():
    kmax = _NR - 1
    delta = ((_R_MAX + 0.0001) / _R_MIN) ** (1.0 / kmax) - 1.0
    r_bins = np.asarray([_R_MIN * (1.0 + delta) ** k for k in range(kmax + 1)],
                        dtype=np.float32)
    angle_bins = np.linspace(-_FOV / 2, _FOV / 2, _NA).astype(np.float32)

    u = np.empty(_NR, dtype=np.float32)
    for k, b in enumerate(r_bins):
        s = np.float32(np.float64(b) ** 2)
        while np.float32(np.sqrt(s)) > b:
            s = np.nextafter(s, np.float32(0), dtype=np.float32)
        while np.float32(np.sqrt(s)) <= b:
            s = np.nextafter(s, np.float32(np.inf), dtype=np.float32)
        u[k] = s

    mg = np.linspace(1.0, 2.0, 200001)
    c2, c1, c0 = np.polyfit(mg, np.log2(mg), 2)
    consts = dict(
        K2=np.float32(1.0 / (2.0 * np.log2(1.0 + delta))),
        K0=np.float32(-np.log2(2.7) / np.log2(1.0 + delta)),
        C2=np.float32(c2), C1=np.float32(c1), C0=np.float32(c0),
        A0=np.float32(angle_bins[0]),
        INVH=np.float32(1.0 / ((float(angle_bins[-1]) - float(angle_bins[0]))
                               / (_NA - 1))),
    )
    return u, consts


_TB_R, _C = _build_tables()


def _angle_table():
    ab = jnp.linspace(-_FOV / 2, _FOV / 2, _NA).astype(jnp.float32)
    return jnp.concatenate(
        [ab, jnp.full((208 - _NA,), jnp.inf, jnp.float32)])


def _tc_body(x_ref, y_ref, z_ref, ang_ref, s_ref, zenc_ref):
    x = x_ref[...]
    y = y_ref[...]
    z = z_ref[...]
    ang = jnp.arctan2(y, x)
    s = x * x + y * y
    radius = jnp.sqrt(s)
    mask = ((jnp.abs(ang) < np.float32(_FOV / 2))
            & (radius < np.float32(_R_MAX))
            & (radius > np.float32(_R_MIN)))
    zf = jnp.floor((z - np.float32(_Z_MIN)) / np.float32(_Z_STEP)).astype(jnp.int32)
    zw = jnp.where(zf < 0, zf + _Z_DEPTH, zf)
    zok = (zw >= 0) & (zw < _Z_DEPTH)
    sidx = pl.program_id(0) // 16
    zenc = jnp.where(mask & zok, (sidx * _Z_DEPTH + zw) * _ROW, _INVALID)
    ang_ref[...] = ang
    s_ref[...] = s
    zenc_ref[...] = zenc.astype(jnp.int32)


def _sc_body(ang_hbm, s_hbm, zenc_hbm, tba_hbm, tbr_hbm, out_hbm,
             zbuf, angc, sc_, zencc, sva, svs, svz, idx1d, val1d, idx2d, val2d,
             tba, tbr, zsem, ssem):
    cid = lax.axis_index("c")
    sid = lax.axis_index("s")
    half_lo = cid * _HALF

    zeros16 = jnp.zeros((16,), jnp.float32)

    def zfill(i, carry):
        zbuf[pl.ds(i * 16, 16)] = zeros16
        return carry

    lax.fori_loop(0, _ZBUF // 16, zfill, 0)
    slab = half_lo + sid * _ZSLAB
    zcopies = [
        pltpu.async_copy(zbuf, out_hbm.at[pl.ds(slab + i * _ZBUF, _ZBUF)], zsem)
        for i in range(_NZDMA)
    ]

    pltpu.sync_copy(tba_hbm, tba)
    pltpu.sync_copy(tbr_hbm, tbr)

    inv16 = jnp.full((16,), _INVALID, jnp.int32)

    def prefill(i, carry):
        svz[pl.ds(i * 16, 16)] = inv16
        return carry

    lax.fori_loop(0, _SURV_CAP // 16, prefill, 0)

    base = sid * _PTS_PER_TILE
    n = jnp.int32(0)
    for chunk in range(_N_CHUNKS):
        cb = base + chunk * _CHUNK
        pltpu.sync_copy(ang_hbm.at[pl.ds(cb, _CHUNK)], angc)
        pltpu.sync_copy(s_hbm.at[pl.ds(cb, _CHUNK)], sc_)
        pltpu.sync_copy(zenc_hbm.at[pl.ds(cb, _CHUNK)], zencc)

        def b1(i, n):
            vz = zencc[pl.ds(i * 16, 16)]
            m = vz < _INVALID
            cnt = plsc.all_reduce_population_count(m)[0]
            nn = jnp.minimum(n, _SURV_CAP - 16)
            plsc.store_compressed(sva.at[pl.ds(nn, 16)],
                                  angc[pl.ds(i * 16, 16)], mask=m)
            plsc.store_compressed(svs.at[pl.ds(nn, 16)],
                                  sc_[pl.ds(i * 16, 16)], mask=m)
            plsc.store_compressed(svz.at[pl.ds(nn, 16)], vz, mask=m)
            return n + cnt

        n = lax.fori_loop(0, _CHUNK // 16, b1, n)

    ones16 = jnp.full((16,), 1.0, jnp.float32)

    def pf2(i, carry):
        e = i * 16 + lax.iota(jnp.int32, 16)
        snk = half_lo + (sid * 7 + e // _NR) * _ROW + e % _NR
        idx1d[pl.ds(i * 16, 16)] = snk
        val1d[pl.ds(i * 16, 16)] = jnp.zeros((16,), jnp.float32)
        return carry

    lax.fori_loop(0, _SURV_CAP // 16, pf2, 0)

    def b2(i, nok):
        a = sva[pl.ds(i * 16, 16)]
        s = svs[pl.ds(i * 16, 16)]
        ze = svz[pl.ds(i * 16, 16)]
        valid = ze < _INVALID

        t = (a - _C["A0"]) * _C["INVH"] + np.float32(0.5)
        jh = jnp.clip(t.astype(jnp.int32), 1, 190)
        cj = jnp.zeros((16,), jnp.int32)
        for d in range(3):
            bv = plsc.load_gather(tba, [jh + (d - 1)])
            cj = cj + (bv < a).astype(jnp.int32)
        jstar = jh - 1 + cj

        bits = plsc.bitcast(s, jnp.int32)
        e = ((bits >> 23) & 0xFF) - 127
        mant = plsc.bitcast((bits & 0x7FFFFF) | 0x3F800000, jnp.float32)
        l2m = _C["C2"] * mant * mant + _C["C1"] * mant + _C["C0"]
        xr = (e.astype(jnp.float32) + l2m) * _C["K2"] + _C["K0"]
        xh = jnp.clip(xr.astype(jnp.int32) + 1, 2, 317)
        cx = jnp.zeros((16,), jnp.int32)
        for d in range(5):
            uv = plsc.load_gather(tbr, [xh + (d - 2)])
            cx = cx + (uv <= s).astype(jnp.int32)
        xstar = xh - 2 + cx

        flat = ze + jstar * _NR + xstar
        ok = valid & (flat >= half_lo) & (flat < half_lo + _HALF)
        cnt = plsc.all_reduce_population_count(ok)[0]
        nn = jnp.minimum(nok, _SURV_CAP - 16)
        plsc.store_compressed(idx1d.at[pl.ds(nn, 16)], flat, mask=ok)
        plsc.store_compressed(val1d.at[pl.ds(nn, 16)], ones16, mask=ok)
        return nok + cnt

    nok = lax.fori_loop(0, _SURV_CAP // 16, b2, jnp.int32(0))

    def rcp(i, carry):
        r = i // 8
        cofs = (i % 8) * 16
        idx2d[r, pl.ds(cofs, 16)] = idx1d[pl.ds(i * 16, 16)]
        val2d[r, pl.ds(cofs, 16)] = val1d[pl.ds(i * 16, 16)]
        return carry

    lax.fori_loop(0, _SURV_CAP // 16, rcp, 0)

    nrows = (nok + 127) // 128
    for cpy in zcopies:
        cpy.wait()
    plsc.subcore_barrier()
    for r in range(16):
        @pl.when(r < nrows)
        def _():
            pltpu.async_copy(val2d.at[r], out_hbm.at[idx2d.at[r]], ssem).wait()


@functools.cache
def _make_sc_scatter():
    mesh = plsc.VectorSubcoreMesh(core_axis_name="c", subcore_axis_name="s")
    return functools.partial(
        pl.kernel,
        out_type=jax.ShapeDtypeStruct((_GRID,), jnp.float32),
        mesh=mesh,
        compiler_params=pltpu.CompilerParams(needs_layout_passes=False, use_tc_tiling_on_sc=True),
        scratch_types=[
            pltpu.VMEM((_ZBUF,), jnp.float32),
            pltpu.VMEM((_CHUNK,), jnp.float32),
            pltpu.VMEM((_CHUNK,), jnp.float32),
            pltpu.VMEM((_CHUNK,), jnp.int32),
            pltpu.VMEM((_SURV_CAP,), jnp.float32),
            pltpu.VMEM((_SURV_CAP,), jnp.float32),
            pltpu.VMEM((_SURV_CAP,), jnp.int32),
            pltpu.VMEM((_SURV_CAP,), jnp.int32),
            pltpu.VMEM((_SURV_CAP,), jnp.float32),
            pltpu.VMEM((16, 128), jnp.int32),
            pltpu.VMEM((16, 128), jnp.float32),
            pltpu.VMEM((208,), jnp.float32),
            pltpu.VMEM((_NR,), jnp.float32),
            pltpu.SemaphoreType.DMA,
            pltpu.SemaphoreType.DMA,
        ],
    )(_sc_body)


def kernel(lidars):
    pts = lidars[0].reshape(_N_PTS, 3).T
    x2 = pts[0].reshape(640, 1024)
    y2 = pts[1].reshape(640, 1024)
    z2 = pts[2].reshape(640, 1024)

    blk = pl.BlockSpec((8, 1024), lambda i: (i, 0))
    ang, s, zenc = pl.pallas_call(
        _tc_body,
        grid=(80,),
        in_specs=[blk, blk, blk],
        out_specs=[blk, blk, blk],
        out_shape=[
            jax.ShapeDtypeStruct((640, 1024), jnp.float32),
            jax.ShapeDtypeStruct((640, 1024), jnp.float32),
            jax.ShapeDtypeStruct((640, 1024), jnp.int32),
        ],
    )(x2, y2, z2)

    out = _make_sc_scatter()(ang.reshape(_N_PTS), s.reshape(_N_PTS),
                             zenc.reshape(_N_PTS),
                             _angle_table(), jnp.asarray(_TB_R))
    return jnp.maximum(out.reshape(500, _NA, _NR), jnp.float32(0.0))


# speedup vs baseline: 1.6130x; 1.6130x over previous
import functools

import numpy as np
import jax
import jax.numpy as jnp
from jax import lax
from jax.experimental import pallas as pl
from jax.experimental.pallas import tpu as pltpu
from jax.experimental.pallas import tpu_sc as plsc

_Z_MIN = -2.0
_Z_STEP = 0.2
_Z_DEPTH = 100
_FOV = 2.268
_NA = 192
_R_MIN = 2.7
_R_MAX = 165.0
_NR = 320

_N_PTS = 5 * 131072
_GRID = 500 * _NA * _NR
_HALF = _GRID // 2
_ROW = _NA * _NR

_INVALID = np.int32(1 << 30)

_PTS_PER_TILE = _N_PTS // 16
_CHUNK = 8192
_N_CHUNKS = _PTS_PER_TILE // _CHUNK
_SURV_CAP = 2048
_ZSLAB = _HALF // 16
_ZBUF = 48000
_NZDMA = _ZSLAB // _ZBUF


def _build_tables():
    kmax = _NR - 1
    delta = ((_R_MAX + 0.0001) / _R_MIN) ** (1.0 / kmax) - 1.0
    r_bins = np.asarray([_R_MIN * (1.0 + delta) ** k for k in range(kmax + 1)],
                        dtype=np.float32)
    angle_bins = np.linspace(-_FOV / 2, _FOV / 2, _NA).astype(np.float32)

    u = np.empty(_NR, dtype=np.float32)
    for k, b in enumerate(r_bins):
        s = np.float32(np.float64(b) ** 2)
        while np.float32(np.sqrt(s)) > b:
            s = np.nextafter(s, np.float32(0), dtype=np.float32)
        while np.float32(np.sqrt(s)) <= b:
            s = np.nextafter(s, np.float32(np.inf), dtype=np.float32)
        u[k] = s

    mg = np.linspace(1.0, 2.0, 200001)
    c2, c1, c0 = np.polyfit(mg, np.log2(mg), 2)
    consts = dict(
        K2=np.float32(1.0 / (2.0 * np.log2(1.0 + delta))),
        K0=np.float32(-np.log2(2.7) / np.log2(1.0 + delta)),
        C2=np.float32(c2), C1=np.float32(c1), C0=np.float32(c0),
        A0=np.float32(angle_bins[0]),
        INVH=np.float32(1.0 / ((float(angle_bins[-1]) - float(angle_bins[0]))
                               / (_NA - 1))),
    )
    return u, consts


_TB_R, _C = _build_tables()


def _angle_table():
    ab = jnp.linspace(-_FOV / 2, _FOV / 2, _NA).astype(jnp.float32)
    return jnp.concatenate(
        [ab, jnp.full((208 - _NA,), jnp.inf, jnp.float32)])


def _tc_body(x_ref, y_ref, z_ref, ang_ref, s_ref, zenc_ref):
    x = x_ref[...]
    y = y_ref[...]
    z = z_ref[...]
    ang = jnp.arctan2(y, x)
    s = x * x + y * y
    radius = jnp.sqrt(s)
    mask = ((jnp.abs(ang) < np.float32(_FOV / 2))
            & (radius < np.float32(_R_MAX))
            & (radius > np.float32(_R_MIN)))
    zf = jnp.floor((z - np.float32(_Z_MIN)) / np.float32(_Z_STEP)).astype(jnp.int32)
    zw = jnp.where(zf < 0, zf + _Z_DEPTH, zf)
    zok = (zw >= 0) & (zw < _Z_DEPTH)
    sidx = pl.program_id(0) // 16
    zenc = jnp.where(mask & zok, (sidx * _Z_DEPTH + zw) * _ROW, _INVALID)
    ang_ref[...] = ang
    s_ref[...] = s
    zenc_ref[...] = zenc.astype(jnp.int32)


def _sc_body(ang_hbm, s_hbm, zenc_hbm, tba_hbm, tbr_hbm, out_hbm,
             angc, sc_, zencc, sva, svs, svz, flt, tbuf, myz, myy, myx,
             slab, tba, tbr, flats_sh):
    cid = lax.axis_index("c")
    sid = lax.axis_index("s")
    zlo = (cid * 250 + jnp.minimum(sid, 10) * 16
           + jnp.maximum(sid - 10, 0) * 15)
    znum = jnp.where(sid < 10, 16, 15)

    pltpu.sync_copy(tba_hbm, tba)
    pltpu.sync_copy(tbr_hbm, tbr)

    inv16 = jnp.full((16,), _INVALID, jnp.int32)

    def prefill(i, carry):
        svz[pl.ds(i * 16, 16)] = inv16
        myz[pl.ds(i * 16, 16)] = jnp.full((16,), -1, jnp.int32)
        return carry

    lax.fori_loop(0, _SURV_CAP // 16, prefill, 0)

    base = sid * _PTS_PER_TILE
    n = jnp.int32(0)
    for chunk in range(_N_CHUNKS):
        cb = base + chunk * _CHUNK
        pltpu.sync_copy(ang_hbm.at[pl.ds(cb, _CHUNK)], angc)
        pltpu.sync_copy(s_hbm.at[pl.ds(cb, _CHUNK)], sc_)
        pltpu.sync_copy(zenc_hbm.at[pl.ds(cb, _CHUNK)], zencc)

        def b1(i, n):
            vz = zencc[pl.ds(i * 16, 16)]
            m = vz < _INVALID
            cnt = plsc.all_reduce_population_count(m)[0]
            nn = jnp.minimum(n, _SURV_CAP - 16)
            plsc.store_compressed(sva.at[pl.ds(nn, 16)],
                                  angc[pl.ds(i * 16, 16)], mask=m)
            plsc.store_compressed(svs.at[pl.ds(nn, 16)],
                                  sc_[pl.ds(i * 16, 16)], mask=m)
            plsc.store_compressed(svz.at[pl.ds(nn, 16)], vz, mask=m)
            return n + cnt

        n = lax.fori_loop(0, _CHUNK // 16, b1, n)

    def b2(i, carry):
        a = sva[pl.ds(i * 16, 16)]
        s = svs[pl.ds(i * 16, 16)]
        ze = svz[pl.ds(i * 16, 16)]
        valid = ze < _INVALID

        t = (a - _C["A0"]) * _C["INVH"] + np.float32(0.5)
        jh = jnp.clip(t.astype(jnp.int32), 1, 190)
        cj = jnp.zeros((16,), jnp.int32)
        for d in range(3):
            bv = plsc.load_gather(tba, [jh + (d - 1)])
            cj = cj + (bv < a).astype(jnp.int32)
        jstar = jh - 1 + cj

        bits = plsc.bitcast(s, jnp.int32)
        e = ((bits >> 23) & 0xFF) - 127
        mant = plsc.bitcast((bits & 0x7FFFFF) | 0x3F800000, jnp.float32)
        l2m = _C["C2"] * mant * mant + _C["C1"] * mant + _C["C0"]
        xr = (e.astype(jnp.float32) + l2m) * _C["K2"] + _C["K0"]
        xh = jnp.clip(xr.astype(jnp.int32) + 1, 2, 317)
        cx = jnp.zeros((16,), jnp.int32)
        for d in range(5):
            uv = plsc.load_gather(tbr, [xh + (d - 2)])
            cx = cx + (uv <= s).astype(jnp.int32)
        xstar = xh - 2 + cx

        flat = ze + jstar * _NR + xstar
        flt[pl.ds(i * 16, 16)] = jnp.where(valid, flat, _INVALID)
        return carry

    lax.fori_loop(0, _SURV_CAP // 16, b2, 0)

    pltpu.sync_copy(flt, flats_sh.at[sid])
    plsc.subcore_barrier()

    myn = jnp.int32(0)
    for t in range(16):
        pltpu.sync_copy(flats_sh.at[t], tbuf)

        def pick(i, myn):
            v = tbuf[pl.ds(i * 16, 16)]
            z = v // _ROW
            m = (z >= zlo) & (z < zlo + znum)
            cnt = plsc.all_reduce_population_count(m)[0]
            nn = jnp.minimum(myn, _SURV_CAP - 16)
            plsc.store_compressed(myz.at[pl.ds(nn, 16)], z, mask=m)
            plsc.store_compressed(myy.at[pl.ds(nn, 16)],
                                  (v // _NR) % _NA, mask=m)
            plsc.store_compressed(myx.at[pl.ds(nn, 16)], v % _NR, mask=m)
            return myn + cnt

        myn = lax.fori_loop(0, _SURV_CAP // 16, pick, myn)

    def zero16(i, carry):
        slab[i // 20, pl.ds((i % 20) * 16, 16)] = jnp.zeros((16,), jnp.float32)
        return carry

    lax.fori_loop(0, _NA * _NR // 16, zero16, 0)

    ones16 = jnp.full((16,), 1.0, jnp.float32)
    zeros16 = jnp.zeros((16,), jnp.float32)
    nvec = (myn + 15) // 16

    def zslice(z, carry):
        def paint(i, carry):
            m = myz[pl.ds(i * 16, 16)] == z
            yv = myy[pl.ds(i * 16, 16)]
            xv = myx[pl.ds(i * 16, 16)]
            plsc.store_scatter(slab, [yv, xv], ones16, mask=m)
            return carry

        lax.fori_loop(0, nvec, paint, 0)
        pltpu.sync_copy(slab, out_hbm.at[z])

        def unpaint(i, carry):
            m = myz[pl.ds(i * 16, 16)] == z
            yv = myy[pl.ds(i * 16, 16)]
            xv = myx[pl.ds(i * 16, 16)]
            plsc.store_scatter(slab, [yv, xv], zeros16, mask=m)
            return carry

        lax.fori_loop(0, nvec, unpaint, 0)
        return carry

    lax.fori_loop(zlo, zlo + znum, zslice, 0)


@functools.cache
def _make_sc_scatter():
    mesh = plsc.VectorSubcoreMesh(core_axis_name="c", subcore_axis_name="s")
    return functools.partial(
        pl.kernel,
        out_type=jax.ShapeDtypeStruct((500, _NA, _NR), jnp.float32),
        mesh=mesh,
        compiler_params=pltpu.CompilerParams(needs_layout_passes=False,
                                             use_tc_tiling_on_sc=True),
        scratch_types=[
            pltpu.VMEM((_CHUNK,), jnp.float32),
            pltpu.VMEM((_CHUNK,), jnp.float32),
            pltpu.VMEM((_CHUNK,), jnp.int32),
            pltpu.VMEM((_SURV_CAP,), jnp.float32),
            pltpu.VMEM((_SURV_CAP,), jnp.float32),
            pltpu.VMEM((_SURV_CAP,), jnp.int32),
            pltpu.VMEM((_SURV_CAP,), jnp.int32),
            pltpu.VMEM((_SURV_CAP,), jnp.int32),
            pltpu.VMEM((_SURV_CAP,), jnp.int32),
            pltpu.VMEM((_SURV_CAP,), jnp.int32),
            pltpu.VMEM((_SURV_CAP,), jnp.int32),
            pltpu.VMEM((_NA, _NR), jnp.float32),
            pltpu.VMEM((208,), jnp.float32),
            pltpu.VMEM((_NR,), jnp.float32),
            pltpu.VMEM_SHARED((16, _SURV_CAP), jnp.int32),
        ],
    )(_sc_body)


def kernel(lidars):
    pts = lidars[0].reshape(_N_PTS, 3).T
    x2 = pts[0].reshape(640, 1024)
    y2 = pts[1].reshape(640, 1024)
    z2 = pts[2].reshape(640, 1024)

    blk = pl.BlockSpec((8, 1024), lambda i: (i, 0))
    ang, s, zenc = pl.pallas_call(
        _tc_body,
        grid=(80,),
        in_specs=[blk, blk, blk],
        out_specs=[blk, blk, blk],
        out_shape=[
            jax.ShapeDtypeStruct((640, 1024), jnp.float32),
            jax.ShapeDtypeStruct((640, 1024), jnp.float32),
            jax.ShapeDtypeStruct((640, 1024), jnp.int32),
        ],
    )(x2, y2, z2)

    return _make_sc_scatter()(ang.reshape(_N_PTS), s.reshape(_N_PTS),
                              zenc.reshape(_N_PTS),
                              _angle_table(), jnp.asarray(_TB_R))


# speedup vs baseline: 1.6684x; 1.0344x over previous
import functools

import numpy as np
import jax
import jax.numpy as jnp
from jax import lax
from jax.experimental import pallas as pl
from jax.experimental.pallas import tpu as pltpu
from jax.experimental.pallas import tpu_sc as plsc

_Z_MIN = -2.0
_Z_STEP = 0.2
_Z_DEPTH = 100
_FOV = 2.268
_NA = 192
_R_MIN = 2.7
_R_MAX = 165.0
_NR = 320

_N_PTS = 5 * 131072
_GRID = 500 * _NA * _NR
_HALF = _GRID // 2
_ROW = _NA * _NR

_INVALID = np.int32(1 << 30)

_PTS_PER_TILE = _N_PTS // 16
_CHUNK = 8192
_N_CHUNKS = _PTS_PER_TILE // _CHUNK
_SURV_CAP = 2048
_ZSLAB = _HALF // 16
_ZBUF = 48000
_NZDMA = _ZSLAB // _ZBUF


def _build_tables():
    kmax = _NR - 1
    delta = ((_R_MAX + 0.0001) / _R_MIN) ** (1.0 / kmax) - 1.0
    r_bins = np.asarray([_R_MIN * (1.0 + delta) ** k for k in range(kmax + 1)],
                        dtype=np.float32)
    angle_bins = np.linspace(-_FOV / 2, _FOV / 2, _NA).astype(np.float32)

    u = np.empty(_NR, dtype=np.float32)
    for k, b in enumerate(r_bins):
        s = np.float32(np.float64(b) ** 2)
        while np.float32(np.sqrt(s)) > b:
            s = np.nextafter(s, np.float32(0), dtype=np.float32)
        while np.float32(np.sqrt(s)) <= b:
            s = np.nextafter(s, np.float32(np.inf), dtype=np.float32)
        u[k] = s

    mg = np.linspace(1.0, 2.0, 200001)
    c2, c1, c0 = np.polyfit(mg, np.log2(mg), 2)
    consts = dict(
        K2=np.float32(1.0 / (2.0 * np.log2(1.0 + delta))),
        K0=np.float32(-np.log2(2.7) / np.log2(1.0 + delta)),
        C2=np.float32(c2), C1=np.float32(c1), C0=np.float32(c0),
        A0=np.float32(angle_bins[0]),
        INVH=np.float32(1.0 / ((float(angle_bins[-1]) - float(angle_bins[0]))
                               / (_NA - 1))),
    )
    return u, consts


_TB_R, _C = _build_tables()


def _angle_table():
    ab = jnp.linspace(-_FOV / 2, _FOV / 2, _NA).astype(jnp.float32)
    return jnp.concatenate(
        [ab, jnp.full((208 - _NA,), jnp.inf, jnp.float32)])


def _tc_body(x_ref, y_ref, z_ref, ang_ref, s_ref, zenc_ref):
    x = x_ref[...]
    y = y_ref[...]
    z = z_ref[...]
    ang = jnp.arctan2(y, x)
    s = x * x + y * y
    radius = jnp.sqrt(s)
    mask = ((jnp.abs(ang) < np.float32(_FOV / 2))
            & (radius < np.float32(_R_MAX))
            & (radius > np.float32(_R_MIN)))
    zf = jnp.floor((z - np.float32(_Z_MIN)) / np.float32(_Z_STEP)).astype(jnp.int32)
    zw = jnp.where(zf < 0, zf + _Z_DEPTH, zf)
    zok = (zw >= 0) & (zw < _Z_DEPTH)
    sidx = pl.program_id(0) // 16
    zenc = jnp.where(mask & zok, (sidx * _Z_DEPTH + zw) * _ROW, _INVALID)
    ang_ref[...] = ang
    s_ref[...] = s
    zenc_ref[...] = zenc.astype(jnp.int32)


def _sc_body(ang_hbm, s_hbm, zenc_hbm, tba_hbm, tbr_hbm, out_hbm,
             angc, sc_, zencc, sva, svs, svz, flt, tbuf, myz, myy, myx,
             slabA, slabB, semA, semB, tba, tbr, flats_sh):
    cid = lax.axis_index("c")
    sid = lax.axis_index("s")
    zlo = (cid * 250 + jnp.minimum(sid, 10) * 16
           + jnp.maximum(sid - 10, 0) * 15)
    znum = jnp.where(sid < 10, 16, 15)

    pltpu.sync_copy(tba_hbm, tba)
    pltpu.sync_copy(tbr_hbm, tbr)

    inv16 = jnp.full((16,), _INVALID, jnp.int32)

    def prefill(i, carry):
        svz[pl.ds(i * 16, 16)] = inv16
        myz[pl.ds(i * 16, 16)] = jnp.full((16,), -1, jnp.int32)
        return carry

    lax.fori_loop(0, _SURV_CAP // 16, prefill, 0)

    base = sid * _PTS_PER_TILE
    n = jnp.int32(0)
    for chunk in range(_N_CHUNKS):
        cb = base + chunk * _CHUNK
        pltpu.sync_copy(ang_hbm.at[pl.ds(cb, _CHUNK)], angc)
        pltpu.sync_copy(s_hbm.at[pl.ds(cb, _CHUNK)], sc_)
        pltpu.sync_copy(zenc_hbm.at[pl.ds(cb, _CHUNK)], zencc)

        def b1(i, n):
            vz = zencc[pl.ds(i * 16, 16)]
            m = vz < _INVALID
            cnt = plsc.all_reduce_population_count(m)[0]
            nn = jnp.minimum(n, _SURV_CAP - 16)
            plsc.store_compressed(sva.at[pl.ds(nn, 16)],
                                  angc[pl.ds(i * 16, 16)], mask=m)
            plsc.store_compressed(svs.at[pl.ds(nn, 16)],
                                  sc_[pl.ds(i * 16, 16)], mask=m)
            plsc.store_compressed(svz.at[pl.ds(nn, 16)], vz, mask=m)
            return n + cnt

        n = lax.fori_loop(0, _CHUNK // 16, b1, n)

    def b2(i, carry):
        a = sva[pl.ds(i * 16, 16)]
        s = svs[pl.ds(i * 16, 16)]
        ze = svz[pl.ds(i * 16, 16)]
        valid = ze < _INVALID

        t = (a - _C["A0"]) * _C["INVH"] + np.float32(0.5)
        jh = jnp.clip(t.astype(jnp.int32), 1, 190)
        cj = jnp.zeros((16,), jnp.int32)
        for d in range(3):
            bv = plsc.load_gather(tba, [jh + (d - 1)])
            cj = cj + (bv < a).astype(jnp.int32)
        jstar = jh - 1 + cj

        bits = plsc.bitcast(s, jnp.int32)
        e = ((bits >> 23) & 0xFF) - 127
        mant = plsc.bitcast((bits & 0x7FFFFF) | 0x3F800000, jnp.float32)
        l2m = _C["C2"] * mant * mant + _C["C1"] * mant + _C["C0"]
        xr = (e.astype(jnp.float32) + l2m) * _C["K2"] + _C["K0"]
        xh = jnp.clip(xr.astype(jnp.int32) + 1, 2, 317)
        cx = jnp.zeros((16,), jnp.int32)
        for d in range(5):
            uv = plsc.load_gather(tbr, [xh + (d - 2)])
            cx = cx + (uv <= s).astype(jnp.int32)
        xstar = xh - 2 + cx

        flat = ze + jstar * _NR + xstar
        flt[pl.ds(i * 16, 16)] = jnp.where(valid, flat, _INVALID)
        return carry

    lax.fori_loop(0, _SURV_CAP // 16, b2, 0)

    pltpu.sync_copy(flt, flats_sh.at[sid])
    plsc.subcore_barrier()

    myn = jnp.int32(0)
    for t in range(16):
        pltpu.sync_copy(flats_sh.at[t], tbuf)

        def pick(i, myn):
            v = tbuf[pl.ds(i * 16, 16)]
            z = v // _ROW
            m = (z >= zlo) & (z < zlo + znum)
            cnt = plsc.all_reduce_population_count(m)[0]
            nn = jnp.minimum(myn, _SURV_CAP - 16)
            plsc.store_compressed(myz.at[pl.ds(nn, 16)], z, mask=m)
            plsc.store_compressed(myy.at[pl.ds(nn, 16)],
                                  (v // _NR) % _NA, mask=m)
            plsc.store_compressed(myx.at[pl.ds(nn, 16)], v % _NR, mask=m)
            return myn + cnt

        myn = lax.fori_loop(0, _SURV_CAP // 16, pick, myn)

    def zero16(i, carry):
        slabA[i // 20, pl.ds((i % 20) * 16, 16)] = jnp.zeros((16,), jnp.float32)
        slabB[i // 20, pl.ds((i % 20) * 16, 16)] = jnp.zeros((16,), jnp.float32)
        return carry

    lax.fori_loop(0, 96 * _NR // 16, zero16, 0)

    ones16 = jnp.full((16,), 1.0, jnp.float32)
    zeros16 = jnp.zeros((16,), jnp.float32)
    nvec = (myn + 15) // 16

    def halfpass(slab, half, z, vals):
        ylo = half * 96

        def paint(i, carry):
            yv = myy[pl.ds(i * 16, 16)]
            m = ((myz[pl.ds(i * 16, 16)] == z)
                 & (yv >= ylo) & (yv < ylo + 96))
            xv = myx[pl.ds(i * 16, 16)]
            plsc.store_scatter(slab, [yv - ylo, xv], vals, mask=m)
            return carry

        lax.fori_loop(0, nvec, paint, 0)

    def zslice(z, carry):
        for half, slab, sem in ((0, slabA, semA), (1, slabB, semB)):
            @pl.when(z > zlo)
            def _():
                pltpu.make_async_copy(
                    slab, out_hbm.at[z - 1, pl.ds(half * 96, 96)], sem).wait()
                halfpass(slab, half, z - 1, zeros16)

            halfpass(slab, half, z, ones16)
            pltpu.async_copy(slab, out_hbm.at[z, pl.ds(half * 96, 96)], sem)
        return carry

    lax.fori_loop(zlo, zlo + znum, zslice, 0)
    pltpu.make_async_copy(
        slabA, out_hbm.at[zlo + znum - 1, pl.ds(0, 96)], semA).wait()
    pltpu.make_async_copy(
        slabB, out_hbm.at[zlo + znum - 1, pl.ds(96, 96)], semB).wait()


@functools.cache
def _make_sc_scatter():
    mesh = plsc.VectorSubcoreMesh(core_axis_name="c", subcore_axis_name="s")
    return functools.partial(
        pl.kernel,
        out_type=jax.ShapeDtypeStruct((500, _NA, _NR), jnp.float32),
        mesh=mesh,
        compiler_params=pltpu.CompilerParams(needs_layout_passes=False,
                                             use_tc_tiling_on_sc=True),
        scratch_types=[
            pltpu.VMEM((_CHUNK,), jnp.float32),
            pltpu.VMEM((_CHUNK,), jnp.float32),
            pltpu.VMEM((_CHUNK,), jnp.int32),
            pltpu.VMEM((_SURV_CAP,), jnp.float32),
            pltpu.VMEM((_SURV_CAP,), jnp.float32),
            pltpu.VMEM((_SURV_CAP,), jnp.int32),
            pltpu.VMEM((_SURV_CAP,), jnp.int32),
            pltpu.VMEM((_SURV_CAP,), jnp.int32),
            pltpu.VMEM((_SURV_CAP,), jnp.int32),
            pltpu.VMEM((_SURV_CAP,), jnp.int32),
            pltpu.VMEM((_SURV_CAP,), jnp.int32),
            pltpu.VMEM((96, _NR), jnp.float32),
            pltpu.VMEM((96, _NR), jnp.float32),
            pltpu.SemaphoreType.DMA,
            pltpu.SemaphoreType.DMA,
            pltpu.VMEM((208,), jnp.float32),
            pltpu.VMEM((_NR,), jnp.float32),
            pltpu.VMEM_SHARED((16, _SURV_CAP), jnp.int32),
        ],
    )(_sc_body)


def kernel(lidars):
    pts = lidars[0].reshape(_N_PTS, 3).T
    x2 = pts[0].reshape(640, 1024)
    y2 = pts[1].reshape(640, 1024)
    z2 = pts[2].reshape(640, 1024)

    blk = pl.BlockSpec((8, 1024), lambda i: (i, 0))
    ang, s, zenc = pl.pallas_call(
        _tc_body,
        grid=(80,),
        in_specs=[blk, blk, blk],
        out_specs=[blk, blk, blk],
        out_shape=[
            jax.ShapeDtypeStruct((640, 1024), jnp.float32),
            jax.ShapeDtypeStruct((640, 1024), jnp.float32),
            jax.ShapeDtypeStruct((640, 1024), jnp.int32),
        ],
    )(x2, y2, z2)

    return _make_sc_scatter()(ang.reshape(_N_PTS), s.reshape(_N_PTS),
                              zenc.reshape(_N_PTS),
                              _angle_table(), jnp.asarray(_TB_R))


# speedup vs baseline: 1.6690x; 1.0004x over previous
import functools

import numpy as np
import jax
import jax.numpy as jnp
from jax import lax
from jax.experimental import pallas as pl
from jax.experimental.pallas import tpu as pltpu
from jax.experimental.pallas import tpu_sc as plsc

_Z_MIN = -2.0
_Z_STEP = 0.2
_Z_DEPTH = 100
_FOV = 2.268
_NA = 192
_R_MIN = 2.7
_R_MAX = 165.0
_NR = 320

_N_PTS = 5 * 131072
_GRID = 500 * _NA * _NR
_HALF = _GRID // 2
_ROW = _NA * _NR

_INVALID = np.int32(1 << 30)

_PTS_PER_TILE = _N_PTS // 16
_CHUNK = 8192
_N_CHUNKS = _PTS_PER_TILE // _CHUNK
_SURV_CAP = 2048


def _build_tables():
    kmax = _NR - 1
    delta = ((_R_MAX + 0.0001) / _R_MIN) ** (1.0 / kmax) - 1.0
    r_bins = np.asarray([_R_MIN * (1.0 + delta) ** k for k in range(kmax + 1)],
                        dtype=np.float32)
    angle_bins = np.linspace(-_FOV / 2, _FOV / 2, _NA).astype(np.float32)

    u = np.empty(_NR, dtype=np.float32)
    for k, b in enumerate(r_bins):
        s = np.float32(np.float64(b) ** 2)
        while np.float32(np.sqrt(s)) > b:
            s = np.nextafter(s, np.float32(0), dtype=np.float32)
        while np.float32(np.sqrt(s)) <= b:
            s = np.nextafter(s, np.float32(np.inf), dtype=np.float32)
        u[k] = s

    mg = np.linspace(1.0, 2.0, 200001)
    c2, c1, c0 = np.polyfit(mg, np.log2(mg), 2)
    consts = dict(
        K2=np.float32(1.0 / (2.0 * np.log2(1.0 + delta))),
        K0=np.float32(-np.log2(2.7) / np.log2(1.0 + delta)),
        C2=np.float32(c2), C1=np.float32(c1), C0=np.float32(c0),
        A0=np.float32(angle_bins[0]),
        INVH=np.float32(1.0 / ((float(angle_bins[-1]) - float(angle_bins[0]))
                               / (_NA - 1))),
    )
    return u, consts


_TB_R, _C = _build_tables()


def _angle_table():
    ab = jnp.linspace(-_FOV / 2, _FOV / 2, _NA).astype(jnp.float32)
    return jnp.concatenate(
        [ab, jnp.full((208 - _NA,), jnp.inf, jnp.float32)])


def _tc_body(x_ref, y_ref, z_ref, ang_ref, s_ref, zenc_ref):
    x = x_ref[...]
    y = y_ref[...]
    z = z_ref[...]
    ang = jnp.arctan2(y, x)
    s = x * x + y * y
    radius = jnp.sqrt(s)
    mask = ((jnp.abs(ang) < np.float32(_FOV / 2))
            & (radius < np.float32(_R_MAX))
            & (radius > np.float32(_R_MIN)))
    zf = jnp.floor((z - np.float32(_Z_MIN)) / np.float32(_Z_STEP)).astype(jnp.int32)
    zw = jnp.where(zf < 0, zf + _Z_DEPTH, zf)
    zok = (zw >= 0) & (zw < _Z_DEPTH)
    sidx = pl.program_id(0) // 16
    zenc = jnp.where(mask & zok, (sidx * _Z_DEPTH + zw) * _ROW, _INVALID)
    ang_ref[...] = ang
    s_ref[...] = s
    zenc_ref[...] = zenc.astype(jnp.int32)


def _sc_body(ang_hbm, s_hbm, zenc_hbm, tba_hbm, tbr_hbm, out_hbm,
             angc, sc_, zencc, sva, svs, svz, flt, tbuf, myz, myy, myx,
             slabA, slabB, semA, semB, tba, tbr, flats_sh):
    cid = lax.axis_index("c")
    sid = lax.axis_index("s")
    zlo = (cid * 250 + jnp.minimum(sid, 10) * 16
           + jnp.maximum(sid - 10, 0) * 15)
    znum = jnp.where(sid < 10, 16, 15)

    pltpu.sync_copy(tba_hbm, tba)
    pltpu.sync_copy(tbr_hbm, tbr)

    inv16 = jnp.full((16,), _INVALID, jnp.int32)

    def prefill(i, carry):
        svz[pl.ds(i * 16, 16)] = inv16
        myz[pl.ds(i * 16, 16)] = jnp.full((16,), -1, jnp.int32)
        return carry

    lax.fori_loop(0, _SURV_CAP // 16, prefill, 0)

    base = sid * _PTS_PER_TILE
    n = jnp.int32(0)
    for chunk in range(_N_CHUNKS):
        cb = base + chunk * _CHUNK
        pltpu.sync_copy(ang_hbm.at[pl.ds(cb, _CHUNK)], angc)
        pltpu.sync_copy(s_hbm.at[pl.ds(cb, _CHUNK)], sc_)
        pltpu.sync_copy(zenc_hbm.at[pl.ds(cb, _CHUNK)], zencc)

        def b1(i, n):
            vz = zencc[pl.ds(i * 16, 16)]
            m = vz < _INVALID
            cnt = plsc.all_reduce_population_count(m)[0]
            nn = jnp.minimum(n, _SURV_CAP - 16)
            plsc.store_compressed(sva.at[pl.ds(nn, 16)],
                                  angc[pl.ds(i * 16, 16)], mask=m)
            plsc.store_compressed(svs.at[pl.ds(nn, 16)],
                                  sc_[pl.ds(i * 16, 16)], mask=m)
            plsc.store_compressed(svz.at[pl.ds(nn, 16)], vz, mask=m)
            return n + cnt

        n = lax.fori_loop(0, _CHUNK // 16, b1, n)

    def b2(i, carry):
        a = sva[pl.ds(i * 16, 16)]
        s = svs[pl.ds(i * 16, 16)]
        ze = svz[pl.ds(i * 16, 16)]
        valid = ze < _INVALID

        t = (a - _C["A0"]) * _C["INVH"] + np.float32(0.5)
        jh = jnp.clip(t.astype(jnp.int32), 1, 190)
        cj = jnp.zeros((16,), jnp.int32)
        for d in range(3):
            bv = plsc.load_gather(tba, [jh + (d - 1)])
            cj = cj + (bv < a).astype(jnp.int32)
        jstar = jh - 1 + cj

        bits = plsc.bitcast(s, jnp.int32)
        e = ((bits >> 23) & 0xFF) - 127
        mant = plsc.bitcast((bits & 0x7FFFFF) | 0x3F800000, jnp.float32)
        l2m = _C["C2"] * mant * mant + _C["C1"] * mant + _C["C0"]
        xr = (e.astype(jnp.float32) + l2m) * _C["K2"] + _C["K0"]
        xh = jnp.clip(xr.astype(jnp.int32) + 1, 2, 317)
        cx = jnp.zeros((16,), jnp.int32)
        for d in range(5):
            uv = plsc.load_gather(tbr, [xh + (d - 2)])
            cx = cx + (uv <= s).astype(jnp.int32)
        xstar = xh - 2 + cx

        flat = ze + jstar * _NR + xstar
        flt[pl.ds(i * 16, 16)] = jnp.where(valid, flat, _INVALID)
        return carry

    lax.fori_loop(0, _SURV_CAP // 16, b2, 0)

    pltpu.sync_copy(flt, flats_sh.at[sid])
    plsc.subcore_barrier()

    myn = jnp.int32(0)
    for t in range(16):
        pltpu.sync_copy(flats_sh.at[t], tbuf)

        def pick(i, myn):
            v = tbuf[pl.ds(i * 16, 16)]
            z = v // _ROW
            m = (z >= zlo) & (z < zlo + znum)
            cnt = plsc.all_reduce_population_count(m)[0]
            nn = jnp.minimum(myn, _SURV_CAP - 16)
            plsc.store_compressed(myz.at[pl.ds(nn, 16)], z, mask=m)
            plsc.store_compressed(myy.at[pl.ds(nn, 16)],
                                  (v // _NR) % _NA, mask=m)
            plsc.store_compressed(myx.at[pl.ds(nn, 16)], v % _NR, mask=m)
            return myn + cnt

        myn = lax.fori_loop(0, _SURV_CAP // 16, pick, myn)

    def zero16(i, carry):
        slabA[i // 20, pl.ds((i % 20) * 16, 16)] = jnp.zeros((16,), jnp.float32)
        slabB[i // 20, pl.ds((i % 20) * 16, 16)] = jnp.zeros((16,), jnp.float32)
        return carry

    lax.fori_loop(0, 96 * _NR // 16, zero16, 0)

    ones16 = jnp.full((16,), 1.0, jnp.float32)
    zeros16 = jnp.zeros((16,), jnp.float32)
    nvec = (myn + 15) // 16

    def halfpass(slab, half, z, vals):
        ylo = half * 96

        def paint(i, carry):
            yv = myy[pl.ds(i * 16, 16)]
            m = ((myz[pl.ds(i * 16, 16)] == z)
                 & (yv >= ylo) & (yv < ylo + 96))
            xv = myx[pl.ds(i * 16, 16)]
            plsc.store_scatter(slab, [yv - ylo, xv], vals, mask=m)
            return carry

        lax.fori_loop(0, nvec, paint, 0)

    def zslice(z, carry):
        for half, slab, sem in ((0, slabA, semA), (1, slabB, semB)):
            @pl.when(z > zlo)
            def _():
                pltpu.make_async_copy(
                    slab, out_hbm.at[z - 1, pl.ds(half * 96, 96)], sem).wait()
                halfpass(slab, half, z - 1, zeros16)

            halfpass(slab, half, z, ones16)
            pltpu.async_copy(slab, out_hbm.at[z, pl.ds(half * 96, 96)], sem)
        return carry

    lax.fori_loop(zlo, zlo + znum, zslice, 0)
    pltpu.make_async_copy(
        slabA, out_hbm.at[zlo + znum - 1, pl.ds(0, 96)], semA).wait()
    pltpu.make_async_copy(
        slabB, out_hbm.at[zlo + znum - 1, pl.ds(96, 96)], semB).wait()


@functools.cache
def _make_sc_scatter():
    mesh = plsc.VectorSubcoreMesh(core_axis_name="c", subcore_axis_name="s")
    return functools.partial(
        pl.kernel,
        out_type=jax.ShapeDtypeStruct((500, _NA, _NR), jnp.float32),
        mesh=mesh,
        compiler_params=pltpu.CompilerParams(needs_layout_passes=False,
                                             use_tc_tiling_on_sc=True),
        scratch_types=[
            pltpu.VMEM((_CHUNK,), jnp.float32),
            pltpu.VMEM((_CHUNK,), jnp.float32),
            pltpu.VMEM((_CHUNK,), jnp.int32),
            pltpu.VMEM((_SURV_CAP,), jnp.float32),
            pltpu.VMEM((_SURV_CAP,), jnp.float32),
            pltpu.VMEM((_SURV_CAP,), jnp.int32),
            pltpu.VMEM((_SURV_CAP,), jnp.int32),
            pltpu.VMEM((_SURV_CAP,), jnp.int32),
            pltpu.VMEM((_SURV_CAP,), jnp.int32),
            pltpu.VMEM((_SURV_CAP,), jnp.int32),
            pltpu.VMEM((_SURV_CAP,), jnp.int32),
            pltpu.VMEM((96, _NR), jnp.float32),
            pltpu.VMEM((96, _NR), jnp.float32),
            pltpu.SemaphoreType.DMA,
            pltpu.SemaphoreType.DMA,
            pltpu.VMEM((208,), jnp.float32),
            pltpu.VMEM((_NR,), jnp.float32),
            pltpu.VMEM_SHARED((16, _SURV_CAP), jnp.int32),
        ],
    )(_sc_body)


def kernel(lidars):
    pts = lidars[0].reshape(_N_PTS, 3).T
    x2 = pts[0].reshape(640, 1024)
    y2 = pts[1].reshape(640, 1024)
    z2 = pts[2].reshape(640, 1024)

    blk = pl.BlockSpec((8, 1024), lambda i: (i, 0))
    ang, s, zenc = pl.pallas_call(
        _tc_body,
        grid=(80,),
        in_specs=[blk, blk, blk],
        out_specs=[blk, blk, blk],
        out_shape=[
            jax.ShapeDtypeStruct((640, 1024), jnp.float32),
            jax.ShapeDtypeStruct((640, 1024), jnp.float32),
            jax.ShapeDtypeStruct((640, 1024), jnp.int32),
        ],
    )(x2, y2, z2)

    return _make_sc_scatter()(ang.reshape(_N_PTS), s.reshape(_N_PTS),
                              zenc.reshape(_N_PTS),
                              _angle_table(), jnp.asarray(_TB_R))


# speedup vs baseline: 1.6930x; 1.0144x over previous
import functools

import numpy as np
import jax
import jax.numpy as jnp
from jax import lax
from jax.experimental import pallas as pl
from jax.experimental.pallas import tpu as pltpu
from jax.experimental.pallas import tpu_sc as plsc

_Z_MIN = -2.0
_Z_STEP = 0.2
_Z_DEPTH = 100
_FOV = 2.268
_NA = 192
_R_MIN = 2.7
_R_MAX = 165.0
_NR = 320

_N_PTS = 5 * 131072
_GRID = 500 * _NA * _NR
_HALF = _GRID // 2
_ROW = _NA * _NR

_INVALID = np.int32(1 << 30)

_PTS_PER_TILE = _N_PTS // 16
_CHUNK = 8192
_N_CHUNKS = _PTS_PER_TILE // _CHUNK
_SURV_CAP = 2048


def _build_tables():
    kmax = _NR - 1
    delta = ((_R_MAX + 0.0001) / _R_MIN) ** (1.0 / kmax) - 1.0
    r_bins = np.asarray([_R_MIN * (1.0 + delta) ** k for k in range(kmax + 1)],
                        dtype=np.float32)
    angle_bins = np.linspace(-_FOV / 2, _FOV / 2, _NA).astype(np.float32)

    u = np.empty(_NR, dtype=np.float32)
    for k, b in enumerate(r_bins):
        s = np.float32(np.float64(b) ** 2)
        while np.float32(np.sqrt(s)) > b:
            s = np.nextafter(s, np.float32(0), dtype=np.float32)
        while np.float32(np.sqrt(s)) <= b:
            s = np.nextafter(s, np.float32(np.inf), dtype=np.float32)
        u[k] = s

    mg = np.linspace(1.0, 2.0, 200001)
    c2, c1, c0 = np.polyfit(mg, np.log2(mg), 2)
    consts = dict(
        K2=np.float32(1.0 / (2.0 * np.log2(1.0 + delta))),
        K0=np.float32(-np.log2(2.7) / np.log2(1.0 + delta)),
        C2=np.float32(c2), C1=np.float32(c1), C0=np.float32(c0),
        A0=np.float32(angle_bins[0]),
        INVH=np.float32(1.0 / ((float(angle_bins[-1]) - float(angle_bins[0]))
                               / (_NA - 1))),
    )
    return u, consts


_TB_R, _C = _build_tables()


def _angle_table():
    ab = jnp.linspace(-_FOV / 2, _FOV / 2, _NA).astype(jnp.float32)
    return jnp.concatenate(
        [ab, jnp.full((208 - _NA,), jnp.inf, jnp.float32)])


def _tc_body(x_ref, y_ref, z_ref, ang_ref, s_ref, zenc_ref):
    x = x_ref[...]
    y = y_ref[...]
    z = z_ref[...]
    ang = jnp.arctan2(y, x)
    s = x * x + y * y
    radius = jnp.sqrt(s)
    mask = ((jnp.abs(ang) < np.float32(_FOV / 2))
            & (radius < np.float32(_R_MAX))
            & (radius > np.float32(_R_MIN)))
    zf = jnp.floor((z - np.float32(_Z_MIN)) / np.float32(_Z_STEP)).astype(jnp.int32)
    zw = jnp.where(zf < 0, zf + _Z_DEPTH, zf)
    zok = (zw >= 0) & (zw < _Z_DEPTH)
    sidx = pl.program_id(0) // 16
    zenc = jnp.where(mask & zok, (sidx * _Z_DEPTH + zw) * _ROW, _INVALID)
    ang_ref[...] = ang
    s_ref[...] = s
    zenc_ref[...] = zenc.astype(jnp.int32)


def _sc_body(ang_hbm, s_hbm, zenc_hbm, tba_hbm, tbr_hbm, out_hbm,
             angc, sc_, zencc, sva, svs, svz, flt, tbuf, myz, myy, myx,
             slabA, slabB, semA, semB, tba, tbr, flats_sh):
    cid = lax.axis_index("c")
    sid = lax.axis_index("s")
    zlo = (cid * 250 + jnp.minimum(sid, 10) * 16
           + jnp.maximum(sid - 10, 0) * 15)
    znum = jnp.where(sid < 10, 16, 15)

    pltpu.sync_copy(tba_hbm, tba)
    pltpu.sync_copy(tbr_hbm, tbr)

    inv16 = jnp.full((16,), _INVALID, jnp.int32)

    def prefill(i, carry):
        svz[pl.ds(i * 16, 16)] = inv16
        myz[pl.ds(i * 16, 16)] = jnp.full((16,), -1, jnp.int32)
        return carry

    lax.fori_loop(0, _SURV_CAP // 16, prefill, 0)

    rbase = sid * (_PTS_PER_TILE // 1024)
    n = jnp.int32(0)
    for chunk in range(_N_CHUNKS):
        r0 = rbase + chunk * (_CHUNK // 1024)
        pltpu.sync_copy(ang_hbm.at[pl.ds(r0, _CHUNK // 1024)], angc)
        pltpu.sync_copy(s_hbm.at[pl.ds(r0, _CHUNK // 1024)], sc_)
        pltpu.sync_copy(zenc_hbm.at[pl.ds(r0, _CHUNK // 1024)], zencc)

        def b1(i, n):
            r = i // 64
            c = (i % 64) * 16
            vz = zencc[r, pl.ds(c, 16)]
            m = vz < _INVALID
            cnt = plsc.all_reduce_population_count(m)[0]
            nn = jnp.minimum(n, _SURV_CAP - 16)
            plsc.store_compressed(sva.at[pl.ds(nn, 16)],
                                  angc[r, pl.ds(c, 16)], mask=m)
            plsc.store_compressed(svs.at[pl.ds(nn, 16)],
                                  sc_[r, pl.ds(c, 16)], mask=m)
            plsc.store_compressed(svz.at[pl.ds(nn, 16)], vz, mask=m)
            return n + cnt

        n = lax.fori_loop(0, _CHUNK // 16, b1, n)

    def b2(i, carry):
        a = sva[pl.ds(i * 16, 16)]
        s = svs[pl.ds(i * 16, 16)]
        ze = svz[pl.ds(i * 16, 16)]
        valid = ze < _INVALID

        t = (a - _C["A0"]) * _C["INVH"] + np.float32(0.5)
        jh = jnp.clip(t.astype(jnp.int32), 1, 190)
        cj = jnp.zeros((16,), jnp.int32)
        for d in range(3):
            bv = plsc.load_gather(tba, [jh + (d - 1)])
            cj = cj + (bv < a).astype(jnp.int32)
        jstar = jh - 1 + cj

        bits = plsc.bitcast(s, jnp.int32)
        e = ((bits >> 23) & 0xFF) - 127
        mant = plsc.bitcast((bits & 0x7FFFFF) | 0x3F800000, jnp.float32)
        l2m = _C["C2"] * mant * mant + _C["C1"] * mant + _C["C0"]
        xr = (e.astype(jnp.float32) + l2m) * _C["K2"] + _C["K0"]
        xh = jnp.clip(xr.astype(jnp.int32) + 1, 2, 317)
        cx = jnp.zeros((16,), jnp.int32)
        for d in range(5):
            uv = plsc.load_gather(tbr, [xh + (d - 2)])
            cx = cx + (uv <= s).astype(jnp.int32)
        xstar = xh - 2 + cx

        flat = ze + jstar * _NR + xstar
        flt[pl.ds(i * 16, 16)] = jnp.where(valid, flat, _INVALID)
        return carry

    lax.fori_loop(0, _SURV_CAP // 16, b2, 0)

    pltpu.sync_copy(flt, flats_sh.at[sid])
    plsc.subcore_barrier()

    myn = jnp.int32(0)
    for t in range(16):
        pltpu.sync_copy(flats_sh.at[t], tbuf)

        def pick(i, myn):
            v = tbuf[pl.ds(i * 16, 16)]
            z = v // _ROW
            m = (z >= zlo) & (z < zlo + znum)
            cnt = plsc.all_reduce_population_count(m)[0]
            nn = jnp.minimum(myn, _SURV_CAP - 16)
            plsc.store_compressed(myz.at[pl.ds(nn, 16)], z, mask=m)
            plsc.store_compressed(myy.at[pl.ds(nn, 16)],
                                  (v // _NR) % _NA, mask=m)
            plsc.store_compressed(myx.at[pl.ds(nn, 16)], v % _NR, mask=m)
            return myn + cnt

        myn = lax.fori_loop(0, _SURV_CAP // 16, pick, myn)

    def zero16(i, carry):
        slabA[i // 20, pl.ds((i % 20) * 16, 16)] = jnp.zeros((16,), jnp.float32)
        slabB[i // 20, pl.ds((i % 20) * 16, 16)] = jnp.zeros((16,), jnp.float32)
        return carry

    lax.fori_loop(0, 96 * _NR // 16, zero16, 0)

    ones16 = jnp.full((16,), 1.0, jnp.float32)
    zeros16 = jnp.zeros((16,), jnp.float32)
    nvec = (myn + 15) // 16

    def halfpass(slab, half, z, vals):
        ylo = half * 96

        def paint(i, carry):
            yv = myy[pl.ds(i * 16, 16)]
            m = ((myz[pl.ds(i * 16, 16)] == z)
                 & (yv >= ylo) & (yv < ylo + 96))
            xv = myx[pl.ds(i * 16, 16)]
            plsc.store_scatter(slab, [yv - ylo, xv], vals, mask=m)
            return carry

        lax.fori_loop(0, nvec, paint, 0)

    def zslice(z, carry):
        for half, slab, sem in ((0, slabA, semA), (1, slabB, semB)):
            @pl.when(z > zlo)
            def _():
                pltpu.make_async_copy(
                    slab, out_hbm.at[z - 1, pl.ds(half * 96, 96)], sem).wait()
                halfpass(slab, half, z - 1, zeros16)

            halfpass(slab, half, z, ones16)
            pltpu.async_copy(slab, out_hbm.at[z, pl.ds(half * 96, 96)], sem)
        return carry

    lax.fori_loop(zlo, zlo + znum, zslice, 0)
    pltpu.make_async_copy(
        slabA, out_hbm.at[zlo + znum - 1, pl.ds(0, 96)], semA).wait()
    pltpu.make_async_copy(
        slabB, out_hbm.at[zlo + znum - 1, pl.ds(96, 96)], semB).wait()


@functools.cache
def _make_sc_scatter():
    mesh = plsc.VectorSubcoreMesh(core_axis_name="c", subcore_axis_name="s")
    return functools.partial(
        pl.kernel,
        out_type=jax.ShapeDtypeStruct((500, _NA, _NR), jnp.float32),
        mesh=mesh,
        compiler_params=pltpu.CompilerParams(needs_layout_passes=False,
                                             use_tc_tiling_on_sc=True),
        scratch_types=[
            pltpu.VMEM((_CHUNK // 1024, 1024), jnp.float32),
            pltpu.VMEM((_CHUNK // 1024, 1024), jnp.float32),
            pltpu.VMEM((_CHUNK // 1024, 1024), jnp.int32),
            pltpu.VMEM((_SURV_CAP,), jnp.float32),
            pltpu.VMEM((_SURV_CAP,), jnp.float32),
            pltpu.VMEM((_SURV_CAP,), jnp.int32),
            pltpu.VMEM((_SURV_CAP,), jnp.int32),
            pltpu.VMEM((_SURV_CAP,), jnp.int32),
            pltpu.VMEM((_SURV_CAP,), jnp.int32),
            pltpu.VMEM((_SURV_CAP,), jnp.int32),
            pltpu.VMEM((_SURV_CAP,), jnp.int32),
            pltpu.VMEM((96, _NR), jnp.float32),
            pltpu.VMEM((96, _NR), jnp.float32),
            pltpu.SemaphoreType.DMA,
            pltpu.SemaphoreType.DMA,
            pltpu.VMEM((208,), jnp.float32),
            pltpu.VMEM((_NR,), jnp.float32),
            pltpu.VMEM_SHARED((16, _SURV_CAP), jnp.int32),
        ],
    )(_sc_body)


def kernel(lidars):
    pts = lidars[0].reshape(_N_PTS, 3).T
    x2 = pts[0].reshape(640, 1024)
    y2 = pts[1].reshape(640, 1024)
    z2 = pts[2].reshape(640, 1024)

    blk = pl.BlockSpec((8, 1024), lambda i: (i, 0))
    ang, s, zenc = pl.pallas_call(
        _tc_body,
        grid=(80,),
        in_specs=[blk, blk, blk],
        out_specs=[blk, blk, blk],
        out_shape=[
            jax.ShapeDtypeStruct((640, 1024), jnp.float32),
            jax.ShapeDtypeStruct((640, 1024), jnp.float32),
            jax.ShapeDtypeStruct((640, 1024), jnp.int32),
        ],
    )(x2, y2, z2)

    return _make_sc_scatter()(ang, s, zenc,
                              _angle_table(), jnp.asarray(_TB_R))
